# baseline (speedup 1.0000x reference)
import functools

import jax
import jax.numpy as jnp
from jax import lax
from jax.experimental import pallas as pl
from jax.experimental.pallas import tpu as pltpu
from jax.experimental.pallas import tpu_sc as plsc

BATCH = 16384
F = 26
D = 16
V = 1000000
NC = 2
NS = 16
NW = NC * NS
EPW = BATCH // NW
QW = 4
TWBLK = 262144

_mesh = plsc.VectorSubcoreMesh(core_axis_name="c", subcore_axis_name="s")


def _tw_body(t_ref, w_ref, o_ref):
    o_ref[...] = jnp.sum(t_ref[...] * w_ref[...], axis=0)


_tw_call = pl.pallas_call(
    _tw_body,
    grid=(pl.cdiv(V, TWBLK),),
    in_specs=[
        pl.BlockSpec((D, TWBLK), lambda i: (0, i)),
        pl.BlockSpec((D, 1), lambda i: (0, 0)),
    ],
    out_specs=pl.BlockSpec((TWBLK,), lambda i: (i,)),
    out_shape=jax.ShapeDtypeStruct((V,), jnp.float32),
)


@functools.partial(
    pl.kernel,
    mesh=_mesh,
    out_type=jax.ShapeDtypeStruct((BATCH,), jnp.float32),
    compiler_params=pltpu.CompilerParams(use_tc_tiling_on_sc=False),
    scratch_types=[
        pltpu.VMEM((F * EPW,), jnp.int32),
        pltpu.VMEM((F * EPW,), jnp.float32),
        pltpu.VMEM((EPW,), jnp.float32),
        pltpu.VMEM((F + 16,), jnp.int32),
        pltpu.VMEM((16,), jnp.float32),
        pltpu.VMEM_SHARED((V,), jnp.float32),
        pltpu.SemaphoreType.DMA,
        pltpu.SemaphoreType.DMA,
    ],
)
def _sc_pool_kernel(xt_hbm, offs_hbm, tw_hbm, b_hbm, out_hbm,
                    xidx, twg, outb, offs_v, b_v, tw_sp, sem, sem_p):
    sid = lax.axis_index("s")
    wid = sid * NC + lax.axis_index("c")

    @pl.when(sid < 8)
    def _():
        sl = pl.ds(sid * (V // 8), V // 8)
        pltpu.async_copy(tw_hbm.at[sl], tw_sp.at[sl], sem_p)

    pltpu.sync_copy(offs_hbm, offs_v)
    pltpu.sync_copy(b_hbm, b_v)
    base = wid * EPW

    def stage_body(f, carry):
        pltpu.async_copy(
            xt_hbm.at[pl.ds(f * BATCH + base, EPW)],
            xidx.at[pl.ds(f * EPW, EPW)],
            sem,
        )
        return carry

    lax.fori_loop(0, F, stage_body, 0)
    pltpu.make_async_copy(xt_hbm.at[pl.ds(0, F * EPW)], xidx, sem).wait()

    def add_body(f, carry):
        off = offs_v[pl.ds(f, 16)][0]

        def vbody(v, c2):
            sl = pl.ds(f * EPW + v * 16, 16)
            xidx[sl] = xidx[sl] + off
            return c2

        lax.fori_loop(0, EPW // 16, vbody, 0)
        return carry

    lax.fori_loop(0, F, add_body, 0)

    @pl.when(sid < 8)
    def _():
        sl = pl.ds(sid * (V // 8), V // 8)
        pltpu.make_async_copy(tw_hbm.at[sl], tw_sp.at[sl], sem_p).wait()

    plsc.subcore_barrier()

    def gather_body(k, carry):
        sl = pl.ds(k * 128, 128)
        pltpu.async_copy(tw_sp.at[xidx.at[sl]], twg.at[sl], sem)
        return carry

    lax.fori_loop(0, F * EPW // 128, gather_body, 0)
    pltpu.make_async_copy(tw_hbm.at[pl.ds(0, F * EPW)], twg, sem).wait()

    bv = b_v[...]

    def pool_body(v, carry):
        acc = twg[pl.ds(v * 16, 16)]
        for f in range(1, F):
            acc = acc + twg[pl.ds(f * EPW + v * 16, 16)]
        z = acc + bv
        outb[pl.ds(v * 16, 16)] = 1.0 / (1.0 + jnp.exp(-z))
        return carry

    lax.fori_loop(0, EPW // 16, pool_body, 0)
    pltpu.sync_copy(outb, out_hbm.at[pl.ds(base, EPW)])


def kernel(x, offsets, emb_table, W, b):
    xt = x.astype(jnp.int32).T.reshape(BATCH * F)
    wv = (W * (1.0 / F)).astype(jnp.float32)
    tw = _tw_call(emb_table.T, wv)
    b16 = jnp.broadcast_to(b.astype(jnp.float32), (16,))
    offs48 = jnp.pad(offsets.astype(jnp.int32), (0, 16))
    return _sc_pool_kernel(xt, offs48, tw, b16)


# speedup vs baseline: 1.0050x; 1.0050x over previous
import functools

import jax
import jax.numpy as jnp
from jax import lax
from jax.experimental import pallas as pl
from jax.experimental.pallas import tpu as pltpu
from jax.experimental.pallas import tpu_sc as plsc

BATCH = 16384
F = 26
D = 16
V = 1000000
NC = 2
NS = 16
NW = NC * NS
EPW = BATCH // NW
QW = 4
TWBLK = 131072

_mesh = plsc.VectorSubcoreMesh(core_axis_name="c", subcore_axis_name="s")


def _tw_body(t_ref, w_ref, o_ref):
    o_ref[...] = jnp.sum(t_ref[...] * w_ref[...], axis=0)


_tw_call = pl.pallas_call(
    _tw_body,
    grid=(pl.cdiv(V, TWBLK),),
    in_specs=[
        pl.BlockSpec((D, TWBLK), lambda i: (0, i)),
        pl.BlockSpec((D, 1), lambda i: (0, 0)),
    ],
    out_specs=pl.BlockSpec((TWBLK,), lambda i: (i,)),
    out_shape=jax.ShapeDtypeStruct((V,), jnp.float32),
)


@functools.partial(
    pl.kernel,
    mesh=_mesh,
    out_type=jax.ShapeDtypeStruct((BATCH,), jnp.float32),
    compiler_params=pltpu.CompilerParams(use_tc_tiling_on_sc=False),
    scratch_types=[
        pltpu.VMEM((F * EPW,), jnp.int32),
        pltpu.VMEM((F * EPW,), jnp.float32),
        pltpu.VMEM((EPW,), jnp.float32),
        pltpu.VMEM((F + 16,), jnp.int32),
        pltpu.VMEM((16,), jnp.float32),
        pltpu.VMEM_SHARED((V,), jnp.float32),
        pltpu.SemaphoreType.DMA,
        pltpu.SemaphoreType.DMA,
    ],
)
def _sc_pool_kernel(xt_hbm, offs_hbm, tw_hbm, b_hbm, out_hbm,
                    xidx, twg, outb, offs_v, b_v, tw_sp, sem, sem_p):
    sid = lax.axis_index("s")
    wid = sid * NC + lax.axis_index("c")

    @pl.when(sid < 8)
    def _():
        sl = pl.ds(sid * (V // 8), V // 8)
        pltpu.async_copy(tw_hbm.at[sl], tw_sp.at[sl], sem_p)

    pltpu.sync_copy(offs_hbm, offs_v)
    pltpu.sync_copy(b_hbm, b_v)
    base = wid * EPW

    def stage_body(f, carry):
        pltpu.async_copy(
            xt_hbm.at[pl.ds(f * BATCH + base, EPW)],
            xidx.at[pl.ds(f * EPW, EPW)],
            sem,
        )
        return carry

    lax.fori_loop(0, F, stage_body, 0)
    pltpu.make_async_copy(xt_hbm.at[pl.ds(0, F * EPW)], xidx, sem).wait()

    def add_body(f, carry):
        off = offs_v[pl.ds(f, 16)][0]

        def vbody(v, c2):
            sl = pl.ds(f * EPW + v * 16, 16)
            xidx[sl] = xidx[sl] + off
            return c2

        lax.fori_loop(0, EPW // 16, vbody, 0)
        return carry

    lax.fori_loop(0, F, add_body, 0)

    @pl.when(sid < 8)
    def _():
        sl = pl.ds(sid * (V // 8), V // 8)
        pltpu.make_async_copy(tw_hbm.at[sl], tw_sp.at[sl], sem_p).wait()

    plsc.subcore_barrier()

    def gather_body(k, carry):
        sl = pl.ds(k * 128, 128)
        pltpu.async_copy(tw_sp.at[xidx.at[sl]], twg.at[sl], sem)
        return carry

    lax.fori_loop(0, F * EPW // 128, gather_body, 0)
    pltpu.make_async_copy(tw_hbm.at[pl.ds(0, F * EPW)], twg, sem).wait()

    bv = b_v[...]

    def pool_body(v, carry):
        acc = twg[pl.ds(v * 16, 16)]
        for f in range(1, F):
            acc = acc + twg[pl.ds(f * EPW + v * 16, 16)]
        z = acc + bv
        outb[pl.ds(v * 16, 16)] = 1.0 / (1.0 + jnp.exp(-z))
        return carry

    lax.fori_loop(0, EPW // 16, pool_body, 0)
    pltpu.sync_copy(outb, out_hbm.at[pl.ds(base, EPW)])


def kernel(x, offsets, emb_table, W, b):
    xt = x.astype(jnp.int32).T.reshape(BATCH * F)
    wv = (W * (1.0 / F)).astype(jnp.float32)
    tw = _tw_call(emb_table.T, wv)
    b16 = jnp.broadcast_to(b.astype(jnp.float32), (16,))
    offs48 = jnp.pad(offsets.astype(jnp.int32), (0, 16))
    return _sc_pool_kernel(xt, offs48, tw, b16)
